# Initial kernel scaffold; baseline (speedup 1.0000x reference)
import jax
import jax.numpy as jnp
from jax.experimental import pallas as pl


def kernel(idx, token_table, pos_table):
    raise NotImplementedError("write your pallas kernel here")


# speedup vs baseline: 1.0584x; 1.0584x over previous
import functools

import jax
import jax.numpy as jnp
from jax import lax
from jax.experimental import pallas as pl
from jax.experimental.pallas import tpu as pltpu
from jax.experimental.pallas import tpu_sc as plsc

NUM_CORES = 2
NUM_SUBCORES = 16
NW = NUM_CORES * NUM_SUBCORES
LANES = 16


def _make_kernel(N, T, V, D, C):
    rows_per_w = N // NW
    n_chunks = rows_per_w // C
    vecs_per_row = D // LANES
    mesh = plsc.VectorSubcoreMesh(core_axis_name="c", subcore_axis_name="s")

    @functools.partial(
        pl.kernel,
        mesh=mesh,
        out_type=jax.ShapeDtypeStruct((N, D), jnp.float32),
        scratch_types=[
            pltpu.VMEM((C,), jnp.int32),
            pltpu.VMEM((C, D), jnp.float32),
            pltpu.VMEM((C, D), jnp.float32),
            pltpu.SemaphoreType.DMA,
        ],
    )
    def k(idx_hbm, tok_hbm, pos_hbm, out_hbm, idx_v, tok_v, rows_v, sem):
        wid = lax.axis_index("s") * NUM_CORES + lax.axis_index("c")
        base = wid * rows_per_w

        def body(ci, carry):
            start = base + ci * C
            t0 = lax.rem(start, T)
            pltpu.sync_copy(idx_hbm.at[pl.ds(start, C)], idx_v)
            gather = pltpu.async_copy(tok_hbm.at[idx_v], tok_v, sem)
            pltpu.sync_copy(pos_hbm.at[pl.ds(t0, C)], rows_v)
            gather.wait()

            def row_body(r, c2):
                for d in range(vecs_per_row):
                    x = tok_v[r, pl.ds(d * LANES, LANES)]
                    plsc.addupdate(rows_v.at[r, pl.ds(d * LANES, LANES)], x)
                return c2

            lax.fori_loop(0, C, row_body, 0)
            pltpu.sync_copy(rows_v, out_hbm.at[pl.ds(start, C)])
            return carry

        lax.fori_loop(0, n_chunks, body, 0)

    return k


def kernel(idx, token_table, pos_table):
    B, T = idx.shape
    V, D = token_table.shape
    N = B * T
    idx_flat = idx.reshape(N).astype(jnp.int32)
    k = _make_kernel(N, T, V, D, C=64)
    out = k(idx_flat, token_table, pos_table)
    return out.reshape(B, T, D)


# speedup vs baseline: 1.0708x; 1.0117x over previous
import functools

import jax
import jax.numpy as jnp
from jax import lax
from jax.experimental import pallas as pl
from jax.experimental.pallas import tpu as pltpu
from jax.experimental.pallas import tpu_sc as plsc

NUM_CORES = 2
NUM_SUBCORES = 16
NW = NUM_CORES * NUM_SUBCORES
LANES = 16
NBUF = 3


def _make_kernel(N, B, T, V, D, C):
    t_per_w = T // NW
    n_tc = t_per_w // C
    n_steps = B * n_tc
    vecs_per_row = D // LANES
    mesh = plsc.VectorSubcoreMesh(core_axis_name="c", subcore_axis_name="s")

    scratch = (
        [pltpu.VMEM((t_per_w, D), jnp.float32)]
        + [pltpu.VMEM((C, D), jnp.float32) for _ in range(NBUF)]
        + [pltpu.VMEM((C,), jnp.int32) for _ in range(NBUF)]
        + [pltpu.SemaphoreType.DMA for _ in range(NBUF)]
    )

    @functools.partial(
        pl.kernel,
        mesh=mesh,
        out_type=jax.ShapeDtypeStruct((N, D), jnp.float32),
        scratch_types=scratch,
    )
    def k(idx_hbm, tok_hbm, pos_hbm, out_hbm, *refs):
        pos_v = refs[0]
        tok_v = refs[1 : 1 + NBUF]
        idx_v = refs[1 + NBUF : 1 + 2 * NBUF]
        sem = refs[1 + 2 * NBUF : 1 + 3 * NBUF]

        wid = lax.axis_index("s") * NUM_CORES + lax.axis_index("c")
        t_base = wid * t_per_w

        def step_start(s):
            b, tc = s // n_tc, s % n_tc
            return b * T + t_base + tc * C

        def fire_gather(s):
            j = s % NBUF
            pltpu.sync_copy(idx_hbm.at[pl.ds(step_start(s), C)], idx_v[j])
            return pltpu.async_copy(tok_hbm.at[idx_v[j]], tok_v[j], sem[j])

        pltpu.sync_copy(pos_hbm.at[pl.ds(t_base, t_per_w)], pos_v)

        gathers = [fire_gather(s) for s in range(min(NBUF, n_steps))]
        stores = [None] * NBUF

        for s in range(n_steps):
            j = s % NBUF
            tc = s % n_tc
            gathers[j].wait()

            def row_body(r, c2, _j=j, _off=tc * C):
                for d in range(vecs_per_row):
                    x = pos_v[_off + r, pl.ds(d * LANES, LANES)]
                    plsc.addupdate(tok_v[_j].at[r, pl.ds(d * LANES, LANES)], x)
                return c2

            lax.fori_loop(0, C, row_body, 0)
            stores[j] = pltpu.async_copy(
                tok_v[j], out_hbm.at[pl.ds(step_start(s), C)], sem[j]
            )
            if s + NBUF < n_steps:
                stores[j].wait()
                gathers[j] = fire_gather(s + NBUF)

        for s in range(max(0, n_steps - NBUF), n_steps):
            stores[s % NBUF].wait()

    return k


def kernel(idx, token_table, pos_table):
    B, T = idx.shape
    V, D = token_table.shape
    N = B * T
    idx_flat = idx.reshape(N).astype(jnp.int32)
    k = _make_kernel(N, B, T, V, D, C=32)
    out = k(idx_flat, token_table, pos_table)
    return out.reshape(B, T, D)


# speedup vs baseline: 1.1411x; 1.0656x over previous
import functools

import jax
import jax.numpy as jnp
from jax import lax
from jax.experimental import pallas as pl
from jax.experimental.pallas import tpu as pltpu
from jax.experimental.pallas import tpu_sc as plsc

NUM_CORES = 2
NUM_SUBCORES = 16
NW = NUM_CORES * NUM_SUBCORES
LANES = 16
NBUF = 3


def _make_kernel(N, B, T, V, D, C):
    t_per_w = T // NW
    n_tc = t_per_w // C
    n_steps = B * n_tc
    vecs_per_row = D // LANES
    mesh = plsc.VectorSubcoreMesh(core_axis_name="c", subcore_axis_name="s")

    scratch = (
        [pltpu.VMEM((t_per_w, D), jnp.float32)]
        + [pltpu.VMEM((B * t_per_w,), jnp.int32)]
        + [pltpu.VMEM((C, D), jnp.float32) for _ in range(NBUF)]
        + [pltpu.SemaphoreType.DMA for _ in range(NBUF)]
    )

    @functools.partial(
        pl.kernel,
        mesh=mesh,
        out_type=jax.ShapeDtypeStruct((N, D), jnp.float32),
        scratch_types=scratch,
    )
    def k(idx_hbm, tok_hbm, pos_hbm, out_hbm, *refs):
        pos_v = refs[0]
        idx_v = refs[1]
        tok_v = refs[2 : 2 + NBUF]
        sem = refs[2 + NBUF : 2 + 2 * NBUF]

        wid = lax.axis_index("s") * NUM_CORES + lax.axis_index("c")
        t_base = wid * t_per_w

        def step_start(s):
            b, tc = s // n_tc, s % n_tc
            return b * T + t_base + tc * C

        def fire_gather(s):
            j = s % NBUF
            return pltpu.async_copy(
                tok_hbm.at[idx_v.at[pl.ds(s * C, C)]], tok_v[j], sem[j]
            )

        for b in range(B):
            pltpu.sync_copy(
                idx_hbm.at[pl.ds(b * T + t_base, t_per_w)],
                idx_v.at[pl.ds(b * t_per_w, t_per_w)],
            )
        pltpu.sync_copy(pos_hbm.at[pl.ds(t_base, t_per_w)], pos_v)

        gathers = [fire_gather(s) for s in range(min(NBUF - 1, n_steps))]
        gathers += [None] * (NBUF - len(gathers))
        stores = [None] * NBUF

        for s in range(n_steps):
            j = s % NBUF
            tc = s % n_tc
            gathers[j].wait()

            def row_body(r, c2, _j=j, _off=tc * C):
                for d in range(vecs_per_row):
                    x = pos_v[_off + r, pl.ds(d * LANES, LANES)]
                    plsc.addupdate(tok_v[_j].at[r, pl.ds(d * LANES, LANES)], x)
                return c2

            lax.fori_loop(0, C, row_body, 0)
            stores[j] = pltpu.async_copy(
                tok_v[j], out_hbm.at[pl.ds(step_start(s), C)], sem[j]
            )
            u = s + NBUF - 1
            if u < n_steps:
                if s >= 1:
                    stores[(s - 1) % NBUF].wait()
                gathers[u % NBUF] = fire_gather(u)

        for s in range(max(0, n_steps - NBUF), n_steps):
            stores[s % NBUF].wait()

    return k


def kernel(idx, token_table, pos_table):
    B, T = idx.shape
    V, D = token_table.shape
    N = B * T
    idx_flat = idx.reshape(N).astype(jnp.int32)
    k = _make_kernel(N, B, T, V, D, C=32)
    out = k(idx_flat, token_table, pos_table)
    return out.reshape(B, T, D)


# speedup vs baseline: 1.3474x; 1.1808x over previous
import functools

import jax
import jax.numpy as jnp
from jax import lax
from jax.experimental import pallas as pl
from jax.experimental.pallas import tpu as pltpu
from jax.experimental.pallas import tpu_sc as plsc

NUM_CORES = 2
NUM_SUBCORES = 16
NW = NUM_CORES * NUM_SUBCORES
LANES = 16


def _make_kernel(N, B, T, V, D, C):
    t_per_w = T // NW
    n_groups = t_per_w // C
    half = (n_groups // 2) * C
    vecs_per_row = D // LANES
    nbuf = 2 * B
    mesh = plsc.VectorSubcoreMesh(core_axis_name="c", subcore_axis_name="s")

    scratch = (
        [pltpu.VMEM((half, D), jnp.float32)]
        + [pltpu.VMEM((B * t_per_w,), jnp.int32)]
        + [pltpu.VMEM((C, D), jnp.float32) for _ in range(nbuf)]
        + [pltpu.SemaphoreType.DMA for _ in range(nbuf)]
        + [pltpu.SemaphoreType.DMA]
    )

    @functools.partial(
        pl.kernel,
        mesh=mesh,
        out_type=jax.ShapeDtypeStruct((N, D), jnp.float32),
        scratch_types=scratch,
    )
    def k(idx_hbm, tok_hbm, pos_hbm, out_hbm, *refs):
        pos_v = refs[0]
        idx_v = refs[1]
        tok_v = refs[2 : 2 + nbuf]
        sem = refs[2 + nbuf : 2 + 2 * nbuf]
        pos_sem = refs[2 + 2 * nbuf]

        wid = lax.axis_index("s") * NUM_CORES + lax.axis_index("c")
        t_base = wid * t_per_w

        def slot(g, b):
            return (g % 2) * B + b

        def fire_gather(g, b):
            j = slot(g, b)
            return pltpu.async_copy(
                tok_hbm.at[idx_v.at[pl.ds(b * t_per_w + g * C, C)]],
                tok_v[j],
                sem[j],
            )

        def fire_pos_load(h):
            return pltpu.async_copy(
                pos_hbm.at[pl.ds(t_base + h * half, half)], pos_v, pos_sem
            )

        for b in range(B):
            pltpu.sync_copy(
                idx_hbm.at[pl.ds(b * T + t_base, t_per_w)],
                idx_v.at[pl.ds(b * t_per_w, t_per_w)],
            )
        pos_load = fire_pos_load(0)
        gathers = {}
        stores = {}
        for g in range(min(2, n_groups)):
            for b in range(B):
                gathers[(g, b)] = fire_gather(g, b)

        for g in range(n_groups):
            if g % 2 == 0:
                pos_load.wait()
            for b in range(B):
                gathers[(g, b)].wait()
            slots = [slot(g, b) for b in range(B)]
            off = (g % 2) * C

            def row_body(r, c2, _slots=slots, _off=off):
                for d in range(vecs_per_row):
                    x = pos_v[_off + r, pl.ds(d * LANES, LANES)]
                    for j in _slots:
                        plsc.addupdate(tok_v[j].at[r, pl.ds(d * LANES, LANES)], x)
                return c2

            lax.fori_loop(0, C, row_body, 0)
            for b in range(B):
                stores[(g, b)] = pltpu.async_copy(
                    tok_v[slot(g, b)],
                    out_hbm.at[pl.ds(b * T + t_base + g * C, C)],
                    sem[slot(g, b)],
                )
            if g == n_groups // 2 - 1:
                pos_load = fire_pos_load(1)
            if g + 2 < n_groups:
                for b in range(B):
                    stores[(g, b)].wait()
                    gathers[(g + 2, b)] = fire_gather(g + 2, b)

        for g in range(max(0, n_groups - 2), n_groups):
            for b in range(B):
                stores[(g, b)].wait()

    return k


def kernel(idx, token_table, pos_table):
    B, T = idx.shape
    V, D = token_table.shape
    N = B * T
    idx_flat = idx.reshape(N).astype(jnp.int32)
    k = _make_kernel(N, B, T, V, D, C=16)
    out = k(idx_flat, token_table, pos_table)
    return out.reshape(B, T, D)
